# baseline (speedup 1.0000x reference)
import functools

import jax
import jax.numpy as jnp
from jax import lax
from jax.experimental import pallas as pl
from jax.experimental.pallas import tpu as pltpu
from jax.experimental.pallas import tpu_sc as plsc

N_ROWS = 10000
D = 256
E = 160000
EPS = 1e-8

NC, NS, L = 2, 16, 16
NW = NC * NS
BLK = 64
EPW = 5120
E_PAD = NW * EPW
NBLK = EPW // BLK


def _norm_body(x_ref, o_ref):
    x = x_ref[...]
    n = jnp.sqrt(jnp.sum(x * x, axis=1, keepdims=True))
    o_ref[...] = x / jnp.maximum(n, EPS)


def _normalize(x):
    return pl.pallas_call(
        _norm_body,
        grid=(5,),
        in_specs=[pl.BlockSpec((2000, D), lambda i: (i, 0))],
        out_specs=pl.BlockSpec((2000, D), lambda i: (i, 0)),
        out_shape=jax.ShapeDtypeStruct((N_ROWS, D), jnp.float32),
    )(x)


@functools.partial(
    pl.kernel,
    out_type=jax.ShapeDtypeStruct((E_PAD,), jnp.float32),
    mesh=plsc.VectorSubcoreMesh(core_axis_name="c", subcore_axis_name="s"),
    compiler_params=pltpu.CompilerParams(
        use_tc_tiling_on_sc=False, needs_layout_passes=False),
    scratch_types=[
        pltpu.VMEM((BLK,), jnp.int32),
        pltpu.VMEM((BLK,), jnp.int32),
        pltpu.VMEM((BLK, D), jnp.float32),
        pltpu.VMEM((BLK, D), jnp.float32),
        pltpu.VMEM((BLK,), jnp.float32),
        pltpu.SemaphoreType.DMA,
        pltpu.SemaphoreType.DMA,
    ],
)
def _sc_dot(u_hbm, v_hbm, src_hbm, dst_hbm, out_hbm,
            idx_u, idx_v, u_rows, v_rows, out_buf, sem_u, sem_v):
    wid = lax.axis_index("s") * NC + lax.axis_index("c")
    lanes = lax.iota(jnp.int32, L)
    zero = jnp.zeros((L,), jnp.float32)

    def block_body(g, carry):
        base = wid * EPW + g * BLK
        pltpu.sync_copy(src_hbm.at[pl.ds(base, BLK)], idx_u)
        pltpu.sync_copy(dst_hbm.at[pl.ds(base, BLK)], idx_v)
        cu = pltpu.async_copy(u_hbm.at[idx_u], u_rows, sem_u)
        cv = pltpu.async_copy(v_hbm.at[idx_v], v_rows, sem_v)
        cu.wait()
        cv.wait()

        def d_body(j, accs):
            res = list(accs)
            for t in range(4):
                d = j * 4 + t
                dcol = jnp.full((L,), d, jnp.int32)
                for grp in range(BLK // L):
                    rows = lanes + grp * L
                    uc = plsc.load_gather(u_rows, [rows, dcol])
                    vc = plsc.load_gather(v_rows, [rows, dcol])
                    res[grp] = res[grp] + uc * vc
            return tuple(res)

        accs = lax.fori_loop(0, D // 4, d_body, (zero,) * (BLK // L))
        for grp in range(BLK // L):
            out_buf[pl.ds(grp * L, L)] = accs[grp]
        pltpu.sync_copy(out_buf, out_hbm.at[pl.ds(base, BLK)])
        return carry

    lax.fori_loop(0, NBLK, block_body, 0)


def kernel(x_user, x_venue, edge_label_index):
    u_n = _normalize(x_user)
    v_n = _normalize(x_venue)
    eli = edge_label_index.astype(jnp.int32)
    pad = jnp.zeros((E_PAD - E,), jnp.int32)
    src = jnp.concatenate([eli[0], pad])
    dst = jnp.concatenate([eli[1], pad])
    out = _sc_dot(u_n, v_n, src, dst)
    return out[:E]


# speedup vs baseline: 1.3290x; 1.3290x over previous
import functools

import jax
import jax.numpy as jnp
from jax import lax
from jax.experimental import pallas as pl
from jax.experimental.pallas import tpu as pltpu
from jax.experimental.pallas import tpu_sc as plsc

N_ROWS = 10000
D = 256
E = 160000
EPS = 1e-8

NC, NS, L = 2, 16, 16
NW = NC * NS
BLK = 64
EPW = 5120
E_PAD = NW * EPW
NBLK = EPW // BLK


def _norm_body(x_ref, o_ref):
    x = x_ref[...]
    n = jnp.sqrt(jnp.sum(x * x, axis=1, keepdims=True))
    o_ref[...] = x / jnp.maximum(n, EPS)


def _normalize(x):
    return pl.pallas_call(
        _norm_body,
        grid=(5,),
        in_specs=[pl.BlockSpec((2000, D), lambda i: (i, 0))],
        out_specs=pl.BlockSpec((2000, D), lambda i: (i, 0)),
        out_shape=jax.ShapeDtypeStruct((N_ROWS, D), jnp.float32),
    )(x)


@functools.partial(
    pl.kernel,
    out_type=jax.ShapeDtypeStruct((E_PAD,), jnp.float32),
    mesh=plsc.VectorSubcoreMesh(core_axis_name="c", subcore_axis_name="s"),
    compiler_params=pltpu.CompilerParams(
        use_tc_tiling_on_sc=False, needs_layout_passes=False),
    scratch_types=[
        pltpu.VMEM((NBLK, BLK), jnp.int32),
        pltpu.VMEM((NBLK, BLK), jnp.int32),
        pltpu.VMEM((BLK, D), jnp.float32),
        pltpu.VMEM((BLK, D), jnp.float32),
        pltpu.VMEM((BLK, D), jnp.float32),
        pltpu.VMEM((BLK, D), jnp.float32),
        pltpu.VMEM((EPW,), jnp.float32),
        pltpu.SemaphoreType.DMA,
        pltpu.SemaphoreType.DMA,
        pltpu.SemaphoreType.DMA,
        pltpu.SemaphoreType.DMA,
    ],
)
def _sc_dot(u_hbm, v_hbm, src_hbm, dst_hbm, out_hbm,
            idx_u, idx_v, u0, u1, v0, v1, out_all, su0, su1, sv0, sv1):
    wid = lax.axis_index("s") * NC + lax.axis_index("c")
    lanes = lax.iota(jnp.int32, L)
    zero = jnp.zeros((L,), jnp.float32)
    ubufs, vbufs = (u0, u1), (v0, v1)
    usems, vsems = (su0, su1), (sv0, sv1)

    pltpu.sync_copy(src_hbm.at[wid], idx_u)
    pltpu.sync_copy(dst_hbm.at[wid], idx_v)

    def fire(g, b):
        pltpu.async_copy(u_hbm.at[idx_u.at[g]], ubufs[b], usems[b])
        pltpu.async_copy(v_hbm.at[idx_v.at[g]], vbufs[b], vsems[b])

    def wait(g, b):
        pltpu.make_async_copy(u_hbm.at[idx_u.at[g]], ubufs[b], usems[b]).wait()
        pltpu.make_async_copy(v_hbm.at[idx_v.at[g]], vbufs[b], vsems[b]).wait()

    fire(0, 0)

    def pair_body(i, carry):
        for b in range(2):
            g = 2 * i + b

            @pl.when(g + 1 < NBLK)
            def _():
                fire(g + 1, 1 - b)

            wait(g, b)
            ur, vr = ubufs[b], vbufs[b]

            def d_body(j, accs):
                res = list(accs)
                for t in range(4):
                    d = j * 4 + t
                    dcol = jnp.full((L,), d, jnp.int32)
                    for grp in range(BLK // L):
                        rows = lanes + grp * L
                        uc = plsc.load_gather(ur, [rows, dcol])
                        vc = plsc.load_gather(vr, [rows, dcol])
                        res[grp] = res[grp] + uc * vc
                return tuple(res)

            accs = lax.fori_loop(0, D // 4, d_body, (zero,) * (BLK // L))
            for grp in range(BLK // L):
                out_all[pl.ds(g * BLK + grp * L, L)] = accs[grp]
        return carry

    lax.fori_loop(0, NBLK // 2, pair_body, 0)
    pltpu.sync_copy(out_all, out_hbm.at[pl.ds(wid * EPW, EPW)])


def kernel(x_user, x_venue, edge_label_index):
    u_n = _normalize(x_user)
    v_n = _normalize(x_venue)
    eli = edge_label_index.astype(jnp.int32)
    pad = jnp.zeros((E_PAD - E,), jnp.int32)
    src = jnp.concatenate([eli[0], pad]).reshape(NW, NBLK, BLK)
    dst = jnp.concatenate([eli[1], pad]).reshape(NW, NBLK, BLK)
    out = _sc_dot(u_n, v_n, src, dst)
    return out[:E]


# speedup vs baseline: 4.1248x; 3.1037x over previous
import functools

import jax
import jax.numpy as jnp
from jax import lax
from jax.experimental import pallas as pl
from jax.experimental.pallas import tpu as pltpu
from jax.experimental.pallas import tpu_sc as plsc

N_ROWS = 10000
D = 256
E = 160000
EPS = 1e-8

NC, NS, L = 2, 16, 16
NW = NC * NS
BLK = 64
EPW = 5120
E_PAD = NW * EPW
NBLK = EPW // BLK


def _norm_body(x_ref, o_ref):
    x = x_ref[...]
    n = jnp.sqrt(jnp.sum(x * x, axis=1, keepdims=True))
    o_ref[...] = x / jnp.maximum(n, EPS)


def _normalize(x):
    return pl.pallas_call(
        _norm_body,
        grid=(5,),
        in_specs=[pl.BlockSpec((2000, D), lambda i: (i, 0))],
        out_specs=pl.BlockSpec((2000, D), lambda i: (i, 0)),
        out_shape=jax.ShapeDtypeStruct((N_ROWS, D), jnp.float32),
    )(x)


@functools.partial(
    pl.kernel,
    out_type=jax.ShapeDtypeStruct((E_PAD,), jnp.float32),
    mesh=plsc.VectorSubcoreMesh(core_axis_name="c", subcore_axis_name="s"),
    compiler_params=pltpu.CompilerParams(
        use_tc_tiling_on_sc=False, needs_layout_passes=False),
    scratch_types=[
        pltpu.VMEM((NBLK, BLK), jnp.int32),
        pltpu.VMEM((NBLK, BLK), jnp.int32),
        pltpu.VMEM((BLK, D), jnp.float32),
        pltpu.VMEM((BLK, D), jnp.float32),
        pltpu.VMEM((BLK, D), jnp.float32),
        pltpu.VMEM((BLK, D), jnp.float32),
        pltpu.VMEM((EPW,), jnp.float32),
        pltpu.SemaphoreType.DMA,
        pltpu.SemaphoreType.DMA,
        pltpu.SemaphoreType.DMA,
        pltpu.SemaphoreType.DMA,
    ],
)
def _sc_dot(u_hbm, v_hbm, src_hbm, dst_hbm, out_hbm,
            idx_u, idx_v, u0, u1, v0, v1, out_all, su0, su1, sv0, sv1):
    wid = lax.axis_index("s") * NC + lax.axis_index("c")
    lanes = lax.iota(jnp.int32, L)
    zero = jnp.zeros((L,), jnp.float32)
    ubufs, vbufs = (u0, u1), (v0, v1)
    usems, vsems = (su0, su1), (sv0, sv1)

    pltpu.sync_copy(src_hbm.at[wid], idx_u)
    pltpu.sync_copy(dst_hbm.at[wid], idx_v)

    def fire(g, b):
        pltpu.async_copy(u_hbm.at[idx_u.at[g]], ubufs[b], usems[b])
        pltpu.async_copy(v_hbm.at[idx_v.at[g]], vbufs[b], vsems[b])

    def wait(g, b):
        pltpu.make_async_copy(u_hbm.at[idx_u.at[g]], ubufs[b], usems[b]).wait()
        pltpu.make_async_copy(v_hbm.at[idx_v.at[g]], vbufs[b], vsems[b]).wait()

    fire(0, 0)

    def pair_body(i, carry):
        for b in range(2):
            g = 2 * i + b

            @pl.when(g + 1 < NBLK)
            def _():
                fire(g + 1, 1 - b)

            wait(g, b)
            ur, vr = ubufs[b], vbufs[b]

            def d_body(j, accs):
                res = list(accs)
                for t in range(4):
                    d = j * 4 + t
                    dcol = (jnp.full((L,), d, jnp.int32) + lanes) & (D - 1)
                    for grp in range(BLK // L):
                        rows = lanes + grp * L
                        uc = plsc.load_gather(ur, [rows, dcol])
                        vc = plsc.load_gather(vr, [rows, dcol])
                        k = grp * 2 + (t & 1)
                        res[k] = res[k] + uc * vc
                return tuple(res)

            accs = lax.fori_loop(0, D // 4, d_body, (zero,) * (2 * BLK // L))
            for grp in range(BLK // L):
                acc = accs[grp * 2] + accs[grp * 2 + 1]
                out_all[pl.ds(g * BLK + grp * L, L)] = acc
        return carry

    lax.fori_loop(0, NBLK // 2, pair_body, 0)
    pltpu.sync_copy(out_all, out_hbm.at[pl.ds(wid * EPW, EPW)])


def kernel(x_user, x_venue, edge_label_index):
    u_n = _normalize(x_user)
    v_n = _normalize(x_venue)
    eli = edge_label_index.astype(jnp.int32)
    pad = jnp.zeros((E_PAD - E,), jnp.int32)
    src = jnp.concatenate([eli[0], pad]).reshape(NW, NBLK, BLK)
    dst = jnp.concatenate([eli[1], pad]).reshape(NW, NBLK, BLK)
    out = _sc_dot(u_n, v_n, src, dst)
    return out[:E]


# speedup vs baseline: 9.5637x; 2.3186x over previous
import functools

import jax
import jax.numpy as jnp
from jax import lax
from jax.experimental import pallas as pl
from jax.experimental.pallas import tpu as pltpu
from jax.experimental.pallas import tpu_sc as plsc

N_ROWS = 10000
NRP = 10240
D = 256
DP = 128
E = 160000
EPS = 1e-8

NC, NS, L = 2, 16, 16
DPH = DP // NC
BLK = 64
NSPLIT = 2
TPW = 10240
E_PAD = NS * TPW
NBLK = TPW // BLK
RPT = NRP // NS


def _norm_body(x_ref, o_ref):
    x = x_ref[...]
    n = jnp.sqrt(jnp.sum(x * x, axis=1, keepdims=True))
    xn = x / jnp.maximum(n, EPS)
    a = jax.lax.bitcast_convert_type(
        xn[:, :DP].astype(jnp.bfloat16), jnp.uint16).astype(jnp.uint32)
    b = jax.lax.bitcast_convert_type(
        xn[:, DP:].astype(jnp.bfloat16), jnp.uint16).astype(jnp.uint32)
    o_ref[...] = jax.lax.bitcast_convert_type(a | (b << 16), jnp.int32)


def _normalize(x):
    return pl.pallas_call(
        _norm_body,
        grid=(5,),
        in_specs=[pl.BlockSpec((2000, D), lambda i: (i, 0))],
        out_specs=pl.BlockSpec((2000, DP), lambda i: (i, 0)),
        out_shape=jax.ShapeDtypeStruct((N_ROWS, DP), jnp.int32),
    )(x)


def _comb_body(p_ref, o_ref):
    o_ref[...] = p_ref[0] + p_ref[1]


def _combine(parts):
    return pl.pallas_call(
        _comb_body,
        grid=(8,),
        in_specs=[pl.BlockSpec((NC, 160, 128), lambda i: (0, i, 0))],
        out_specs=pl.BlockSpec((160, 128), lambda i: (i, 0)),
        out_shape=jax.ShapeDtypeStruct((E_PAD // 128, 128), jnp.float32),
    )(parts)


@functools.partial(
    pl.kernel,
    out_type=jax.ShapeDtypeStruct((NC, E_PAD), jnp.float32),
    mesh=plsc.VectorSubcoreMesh(core_axis_name="c", subcore_axis_name="s"),
    compiler_params=pltpu.CompilerParams(
        use_tc_tiling_on_sc=False, needs_layout_passes=False),
    scratch_types=[
        pltpu.VMEM((NBLK * NSPLIT, BLK // NSPLIT), jnp.int32),
        pltpu.VMEM((NBLK * NSPLIT, BLK // NSPLIT), jnp.int32),
        pltpu.VMEM_SHARED((NRP, DPH), jnp.int32),
        pltpu.VMEM_SHARED((NRP, DPH), jnp.int32),
        pltpu.VMEM((BLK, DPH), jnp.int32),
        pltpu.VMEM((BLK, DPH), jnp.int32),
        pltpu.VMEM((BLK, DPH), jnp.int32),
        pltpu.VMEM((BLK, DPH), jnp.int32),
        pltpu.VMEM((TPW,), jnp.float32),
        pltpu.SemaphoreType.DMA,
        pltpu.SemaphoreType.DMA,
        pltpu.SemaphoreType.DMA,
        pltpu.SemaphoreType.DMA,
    ],
)
def _sc_dot(u_hbm, v_hbm, src_hbm, dst_hbm, out_hbm,
            idx_u, idx_v, u_sp, v_sp, u0, u1, v0, v1, out_all,
            su0, su1, sv0, sv1):
    cid = lax.axis_index("c")
    sid = lax.axis_index("s")
    lanes = lax.iota(jnp.int32, L)
    zero = jnp.zeros((L,), jnp.float32)
    ubufs, vbufs = (u0, u1), (v0, v1)
    usems, vsems = (su0, su1), (sv0, sv1)

    rs = pl.ds(sid * RPT, RPT)
    pltpu.sync_copy(u_hbm.at[cid, rs], u_sp.at[rs])
    pltpu.sync_copy(v_hbm.at[cid, rs], v_sp.at[rs])
    pltpu.sync_copy(src_hbm.at[sid], idx_u)
    pltpu.sync_copy(dst_hbm.at[sid], idx_v)
    plsc.subcore_barrier()

    def fire(g, b):
        for s in range(NSPLIT):
            sl = pl.ds(s * (BLK // NSPLIT), BLK // NSPLIT)
            pltpu.async_copy(
                u_sp.at[idx_u.at[g * NSPLIT + s]], ubufs[b].at[sl], usems[b])
            pltpu.async_copy(
                v_sp.at[idx_v.at[g * NSPLIT + s]], vbufs[b].at[sl], vsems[b])

    def wait(g, b):
        for s in range(NSPLIT):
            sl = pl.ds(s * (BLK // NSPLIT), BLK // NSPLIT)
            pltpu.make_async_copy(
                u_sp.at[idx_u.at[g * NSPLIT + s]], ubufs[b].at[sl],
                usems[b]).wait()
            pltpu.make_async_copy(
                v_sp.at[idx_v.at[g * NSPLIT + s]], vbufs[b].at[sl],
                vsems[b]).wait()

    fire(0, 0)

    def pair_body(i, carry):
        for b in range(2):
            g = 2 * i + b

            @pl.when(g + 1 < NBLK)
            def _():
                fire(g + 1, 1 - b)

            wait(g, b)
            ur, vr = ubufs[b], vbufs[b]

            def d_body(j, accs):
                res = list(accs)
                for t in range(4):
                    d = j * 4 + t
                    dcol = (jnp.full((L,), d, jnp.int32) + lanes) & (DPH - 1)
                    for grp in range(BLK // L):
                        rows = lanes + grp * L
                        uc = plsc.load_gather(ur, [rows, dcol])
                        vc = plsc.load_gather(vr, [rows, dcol])
                        ua, ub = plsc.unpack(
                            plsc.bitcast(uc, jnp.bfloat16),
                            format=plsc.PackFormat.INTERLEAVED)
                        va, vb = plsc.unpack(
                            plsc.bitcast(vc, jnp.bfloat16),
                            format=plsc.PackFormat.INTERLEAVED)
                        k = grp * 2 + (t & 1)
                        res[k] = res[k] + ua * va + ub * vb
                return tuple(res)

            accs = lax.fori_loop(0, DPH // 4, d_body, (zero,) * (2 * BLK // L))
            for grp in range(BLK // L):
                acc = accs[grp * 2] + accs[grp * 2 + 1]
                out_all[pl.ds(g * BLK + grp * L, L)] = acc
        return carry

    lax.fori_loop(0, NBLK // 2, pair_body, 0)
    pltpu.sync_copy(out_all, out_hbm.at[cid, pl.ds(sid * TPW, TPW)])


def kernel(x_user, x_venue, edge_label_index):
    u_pk = _normalize(x_user)
    v_pk = _normalize(x_venue)
    rpad = ((0, NRP - N_ROWS), (0, 0), (0, 0))
    u3 = jnp.pad(u_pk.reshape(N_ROWS, NC, DPH).transpose(1, 0, 2),
                 ((0, 0),) + rpad[:2])
    v3 = jnp.pad(v_pk.reshape(N_ROWS, NC, DPH).transpose(1, 0, 2),
                 ((0, 0),) + rpad[:2])
    eli = edge_label_index.astype(jnp.int32)
    pad = jnp.zeros((E_PAD - E,), jnp.int32)
    src = jnp.concatenate([eli[0], pad]).reshape(NS, NBLK * NSPLIT,
                                                 BLK // NSPLIT)
    dst = jnp.concatenate([eli[1], pad]).reshape(NS, NBLK * NSPLIT,
                                                 BLK // NSPLIT)
    parts = _sc_dot(u3, v3, src, dst)
    out = _combine(parts.reshape(NC, E_PAD // 128, 128))
    return out.reshape(E_PAD)[:E]
